# baseline (speedup 1.0000x reference)
import jax
import jax.numpy as jnp
from jax import lax
from jax.experimental import pallas as pl
from jax.experimental.pallas import tpu as pltpu
from jax.experimental.pallas import tpu_sc as plsc

B, C, H, W = 8, 192, 224, 224
C_BLK = 64
C_PAD = 256


def _sc_gather(idx_hbm, table_hbm, out_hbm, idx_v, rows_v, sem):
    wid = lax.axis_index("s") * 2 + lax.axis_index("c")

    @pl.when(wid == 0)
    def _():
        pltpu.sync_copy(idx_hbm, idx_v)
        pltpu.async_copy(table_hbm.at[idx_v], rows_v, sem).wait()
        pltpu.sync_copy(rows_v, out_hbm)


@jax.jit
def _gather_rows(cmd_idx, cmd_embedding):
    mesh = plsc.VectorSubcoreMesh(core_axis_name="c", subcore_axis_name="s")
    return pl.kernel(
        _sc_gather,
        out_type=jax.ShapeDtypeStruct((B, C_PAD), jnp.float32),
        mesh=mesh,
        scratch_types=[
            pltpu.VMEM((B,), jnp.int32),
            pltpu.VMEM((B, C_PAD), jnp.float32),
            pltpu.SemaphoreType.DMA,
        ],
    )(cmd_idx, cmd_embedding)


BC = B * C
HL, WL = 392, 128
K = 16
NBUF = 4
N_CHUNKS = BC // K


def _pipe_kernel(emb_smem, x_hbm, o_hbm, in_buf, out_buf, in_sem, out_sem):
    i = pl.program_id(0)
    slot = lax.rem(i, NBUF)

    def in_copy(ci, sl):
        return pltpu.make_async_copy(
            x_hbm.at[pl.ds(ci * K, K)], in_buf.at[sl], in_sem.at[sl])

    def out_copy(ci, sl):
        return pltpu.make_async_copy(
            out_buf.at[sl], o_hbm.at[pl.ds(ci * K, K)], out_sem.at[sl])

    @pl.when(i == 0)
    def _():
        for j in range(NBUF):
            in_copy(j, j).start()

    in_copy(i, slot).wait()

    @pl.when(i >= NBUF)
    def _():
        out_copy(i - NBUF, slot).wait()

    for r in range(K):
        ch = i * K + r
        out_buf[slot, r] = in_buf[slot, r] + emb_smem[ch // C, lax.rem(ch, C)]

    out_copy(i, slot).start()

    @pl.when(i + NBUF < N_CHUNKS)
    def _():
        in_copy(i + NBUF, slot).start()

    @pl.when(i == N_CHUNKS - 1)
    def _():
        for j in range(NBUF - 1):
            ci = N_CHUNKS - NBUF + j
            out_copy(ci, lax.rem(ci, NBUF)).wait()
        out_copy(i, slot).wait()


@jax.jit
def _broadcast_add(x3, emb):
    return pl.pallas_call(
        _pipe_kernel,
        grid=(N_CHUNKS,),
        in_specs=[
            pl.BlockSpec(memory_space=pltpu.SMEM),
            pl.BlockSpec(memory_space=pl.ANY),
        ],
        out_specs=pl.BlockSpec(memory_space=pl.ANY),
        out_shape=jax.ShapeDtypeStruct((BC, HL, WL), jnp.float32),
        scratch_shapes=[
            pltpu.VMEM((NBUF, K, HL, WL), jnp.float32),
            pltpu.VMEM((NBUF, K, HL, WL), jnp.float32),
            pltpu.SemaphoreType.DMA((NBUF,)),
            pltpu.SemaphoreType.DMA((NBUF,)),
        ],
        compiler_params=pltpu.CompilerParams(
            dimension_semantics=("arbitrary",)),
    )(emb, x3)


def kernel(x, cmd_idx, cmd_embedding):
    table = jnp.pad(cmd_embedding, ((0, 0), (0, C_PAD - C)))
    emb = _gather_rows(cmd_idx.astype(jnp.int32), table)
    out3 = _broadcast_add(x.reshape(BC, HL, WL), emb)
    return out3.reshape(B, C, H, W)


# speedup vs baseline: 4.1464x; 4.1464x over previous
import jax
import jax.numpy as jnp
from jax import lax
from jax.experimental import pallas as pl
from jax.experimental.pallas import tpu as pltpu
from jax.experimental.pallas import tpu_sc as plsc

B, C, H, W = 8, 192, 224, 224
C_BLK = 64
C_PAD = 256


def _sc_gather(idx_hbm, table_hbm, out_hbm, idx_v, rows_v, sem):
    wid = lax.axis_index("s") * 2 + lax.axis_index("c")

    @pl.when(wid == 0)
    def _():
        pltpu.sync_copy(idx_hbm, idx_v)
        pltpu.async_copy(table_hbm.at[idx_v], rows_v, sem).wait()
        pltpu.sync_copy(rows_v, out_hbm)


@jax.jit
def _gather_rows(cmd_idx, cmd_embedding):
    mesh = plsc.VectorSubcoreMesh(core_axis_name="c", subcore_axis_name="s")
    return pl.kernel(
        _sc_gather,
        out_type=jax.ShapeDtypeStruct((B, C_PAD), jnp.float32),
        mesh=mesh,
        scratch_types=[
            pltpu.VMEM((B,), jnp.int32),
            pltpu.VMEM((B, C_PAD), jnp.float32),
            pltpu.SemaphoreType.DMA,
        ],
    )(cmd_idx, cmd_embedding)


BC = B * C
HL, WL = H, W
K = 32
NBUF = 4
N_CHUNKS = BC // K


def _pipe_kernel(emb_smem, x_hbm, o_hbm, in_buf, out_buf, in_sem, out_sem):
    i = pl.program_id(0)
    slot = lax.rem(i, NBUF)

    def in_copy(ci, sl):
        return pltpu.make_async_copy(
            x_hbm.at[pl.ds(ci * K, K)], in_buf.at[sl], in_sem.at[sl])

    def out_copy(ci, sl):
        return pltpu.make_async_copy(
            out_buf.at[sl], o_hbm.at[pl.ds(ci * K, K)], out_sem.at[sl])

    @pl.when(i == 0)
    def _():
        for j in range(NBUF):
            in_copy(j, j).start()

    in_copy(i, slot).wait()

    @pl.when(i >= NBUF)
    def _():
        out_copy(i - NBUF, slot).wait()

    for r in range(K):
        ch = i * K + r
        out_buf[slot, r] = in_buf[slot, r] + emb_smem[ch // C, lax.rem(ch, C)]

    out_copy(i, slot).start()

    @pl.when(i + NBUF < N_CHUNKS)
    def _():
        in_copy(i + NBUF, slot).start()

    @pl.when(i == N_CHUNKS - 1)
    def _():
        for j in range(NBUF - 1):
            ci = N_CHUNKS - NBUF + j
            out_copy(ci, lax.rem(ci, NBUF)).wait()
        out_copy(i, slot).wait()


@jax.jit
def _broadcast_add(x3, emb):
    return pl.pallas_call(
        _pipe_kernel,
        grid=(N_CHUNKS,),
        in_specs=[
            pl.BlockSpec(memory_space=pltpu.SMEM),
            pl.BlockSpec(memory_space=pl.ANY),
        ],
        out_specs=pl.BlockSpec(memory_space=pl.ANY),
        out_shape=jax.ShapeDtypeStruct((BC, HL, WL), jnp.float32),
        scratch_shapes=[
            pltpu.VMEM((NBUF, K, HL, WL), jnp.float32),
            pltpu.VMEM((NBUF, K, HL, WL), jnp.float32),
            pltpu.SemaphoreType.DMA((NBUF,)),
            pltpu.SemaphoreType.DMA((NBUF,)),
        ],
        compiler_params=pltpu.CompilerParams(
            dimension_semantics=("arbitrary",)),
    )(emb, x3)


def kernel(x, cmd_idx, cmd_embedding):
    table = jnp.pad(cmd_embedding, ((0, 0), (0, C_PAD - C)))
    emb = _gather_rows(cmd_idx.astype(jnp.int32), table)
    out3 = _broadcast_add(x.reshape(BC, HL, WL), emb)
    return out3.reshape(B, C, H, W)
